# Initial kernel scaffold; baseline (speedup 1.0000x reference)
import jax
import jax.numpy as jnp
from jax.experimental import pallas as pl


def kernel(media, cu_seqlens, Wa, ba, Wg, bg, Ww, bw, W1, b1, W2, b2, W3, b3, output_scale, output_bias):
    raise NotImplementedError("write your pallas kernel here")


# speedup vs baseline: 2.3968x; 2.3968x over previous
import functools

import jax
import jax.numpy as jnp
from jax import lax
from jax.experimental import pallas as pl
from jax.experimental.pallas import tpu as pltpu

_NEG = -1e30


def _fused_body(cu_ref, media_ref, WaT_ref, ba_ref, WgT_ref, bg_ref, WwT_ref,
                bw_ref, W1T_ref, b1_ref, W2T_ref, b2_ref, W3T_ref, b3_ref,
                out_ref, m_ref, z_ref, o_ref, *, blk, nsteps, nseg, ncls):
    i = pl.program_id(0)

    @pl.when(i == 0)
    def _init():
        m_ref[...] = jnp.full((ncls, nseg), _NEG, jnp.float32)
        z_ref[...] = jnp.zeros((ncls, nseg), jnp.float32)
        o_ref[...] = jnp.zeros((ncls, nseg), jnp.float32)

    x = media_ref[...]
    a = jnp.tanh(jnp.dot(x, WaT_ref[...], preferred_element_type=jnp.float32)
                 + ba_ref[...])
    g = jax.nn.sigmoid(jnp.dot(x, WgT_ref[...],
                               preferred_element_type=jnp.float32)
                       + bg_ref[...])
    s = jnp.dot(a * g, WwT_ref[...], preferred_element_type=jnp.float32) \
        + bw_ref[...]
    h1 = jax.nn.gelu(jnp.dot(x, W1T_ref[...],
                             preferred_element_type=jnp.float32) + b1_ref[...])
    h2 = jax.nn.gelu(jnp.dot(h1, W2T_ref[...],
                             preferred_element_type=jnp.float32) + b2_ref[...])
    logit = jnp.dot(h2, W3T_ref[...], preferred_element_type=jnp.float32) \
        + b3_ref[...]

    tok = i * blk + lax.broadcasted_iota(jnp.int32, (blk, nseg), 0)
    ids = jnp.zeros((blk, nseg), jnp.int32)
    for j in range(1, nseg + 1):
        ids = ids + jnp.where(tok >= cu_ref[j], 1, 0)
    segcol = lax.broadcasted_iota(jnp.int32, (blk, nseg), 1)
    onehot = (ids == segcol).astype(jnp.float32)

    m_old = m_ref[...]
    m_parts = []
    for c in range(ncls):
        sc = s[:, c:c + 1]
        masked = jnp.where(onehot > 0, jnp.broadcast_to(sc, (blk, nseg)), _NEG)
        m_parts.append(jnp.max(masked, axis=0, keepdims=True))
    m_blk = jnp.concatenate(m_parts, axis=0)
    m_new = jnp.maximum(m_old, m_blk)
    corr = jnp.exp(m_old - m_new)

    z_parts, o_parts = [], []
    for c in range(ncls):
        mrow = m_new[c:c + 1, :]
        gm = jnp.sum(onehot * mrow, axis=1, keepdims=True)
        e = jnp.exp(s[:, c:c + 1] - gm)
        z_parts.append(jnp.sum(onehot * e, axis=0, keepdims=True))
        o_parts.append(jnp.sum(onehot * (e * logit[:, c:c + 1]),
                               axis=0, keepdims=True))
    z_blk = jnp.concatenate(z_parts, axis=0)
    o_blk = jnp.concatenate(o_parts, axis=0)

    m_ref[...] = m_new
    z_ref[...] = z_ref[...] * corr + z_blk
    o_ref[...] = o_ref[...] * corr + o_blk

    @pl.when(i == nsteps - 1)
    def _fin():
        z = z_ref[...]
        o = o_ref[...]
        out_ref[...] = jnp.where(z > 0, o / z, 0.0)


def kernel(media, cu_seqlens, Wa, ba, Wg, bg, Ww, bw, W1, b1, W2, b2, W3, b3,
           output_scale, output_bias):
    n_tok, d = media.shape
    nseg = cu_seqlens.shape[0] - 1
    ncls = Ww.shape[0]
    h = Wa.shape[0]
    d1 = W1.shape[0]
    d2 = W2.shape[0]
    blk = 2048
    nsteps = n_tok // blk

    body = functools.partial(_fused_body, blk=blk, nsteps=nsteps, nseg=nseg,
                             ncls=ncls)
    row = lambda v: v.reshape(1, -1)
    const = lambda shape: pl.BlockSpec(shape, lambda i: (0, 0))
    out = pl.pallas_call(
        body,
        grid=(nsteps,),
        in_specs=[
            pl.BlockSpec(memory_space=pltpu.SMEM),
            pl.BlockSpec((blk, d), lambda i: (i, 0)),
            const((d, h)), const((1, h)),
            const((d, h)), const((1, h)),
            const((d, ncls)), const((1, ncls)),
            const((d, d1)), const((1, d1)),
            const((d1, d2)), const((1, d2)),
            const((d2, ncls)), const((1, ncls)),
        ],
        out_specs=pl.BlockSpec((ncls, nseg), lambda i: (0, 0)),
        out_shape=jax.ShapeDtypeStruct((ncls, nseg), jnp.float32),
        scratch_shapes=[pltpu.VMEM((ncls, nseg), jnp.float32)] * 3,
    )(cu_seqlens, media, Wa.T, row(ba), Wg.T, row(bg), Ww.T, row(bw),
      W1.T, row(b1), W2.T, row(b2), W3.T, row(b3))
    return out.T * output_scale + output_bias
